# Initial kernel scaffold; baseline (speedup 1.0000x reference)
import jax
import jax.numpy as jnp
from jax.experimental import pallas as pl


def kernel(x, embedding_weight):
    raise NotImplementedError("write your pallas kernel here")


# speedup vs baseline: 5.7254x; 5.7254x over previous
import jax
import jax.numpy as jnp
from jax.experimental import pallas as pl

B, L, D = 16384, 200, 128
BB = 64


def _body(x_ref, w_ref, o_ref):
    xf = x_ref[...].astype(jnp.float32)
    w0 = w_ref[0, :].reshape(1, D, 1)
    dw = (w_ref[1, :] - w_ref[0, :]).reshape(1, D, 1)
    o_ref[...] = w0 + dw * xf[:, None, :]


def kernel(x, embedding_weight):
    x = x.astype(jnp.int32)
    return pl.pallas_call(
        _body,
        grid=(B // BB,),
        in_specs=[
            pl.BlockSpec((BB, L), lambda i: (i, 0)),
            pl.BlockSpec((2, D), lambda i: (0, 0)),
        ],
        out_specs=pl.BlockSpec((BB, D, L), lambda i: (i, 0, 0)),
        out_shape=jax.ShapeDtypeStruct((B, D, L), jnp.float32),
    )(x, embedding_weight)
